# baseline (speedup 1.0000x reference)
import functools

import jax
import jax.numpy as jnp
from jax import lax
from jax.experimental import pallas as pl
from jax.experimental.pallas import tpu as pltpu
from jax.experimental.pallas import tpu_sc as plsc

N = 10000
E = 320000
F_IN = 128

NC = 2
NS = 16
NW = NC * NS
NLOC = 320
EPT = E // NW
BSTR = 528
BCAP = 512

_SDS = jax.ShapeDtypeStruct
_mesh = plsc.VectorSubcoreMesh(core_axis_name="c", subcore_axis_name="s")


def _wid():
    return lax.axis_index("s") * NC + lax.axis_index("c")


def _lane0(v):
    return lax.squeeze(lax.slice(v, (0,), (1,)), (0,))


def _splat(s):
    return jnp.full((16,), s, jnp.int32)


def _bin_body(ef_hbm, souts_hbm, douts_hbm, cnts_hbm,
              sbuf, dbuf, so_v, do_v, cn_v):
    w = _wid()
    e0 = w * EPT

    pltpu.sync_copy(ef_hbm.at[pl.ds(e0, EPT)], sbuf.at[pl.ds(0, EPT)])
    pltpu.sync_copy(ef_hbm.at[pl.ds(E + e0, EPT)], dbuf.at[pl.ds(0, EPT)])

    def z16(i, _):
        cn_v[pl.ds(i * 16, 16)] = jnp.zeros((16,), jnp.int32)
        return 0
    lax.fori_loop(0, NW, z16, 0)

    def zs(i, _):
        so_v[pl.ds(i * 16, 16)] = jnp.zeros((16,), jnp.int32)
        return 0
    lax.fori_loop(0, NW * BSTR // 16, zs, 0)

    def edge(e, _):
        d = _lane0(dbuf[pl.ds(e, 16)])
        s = _lane0(sbuf[pl.ds(e, 16)])
        b = (d * 6554) >> 21
        cb = jnp.minimum(_lane0(cn_v[pl.ds(b * 16, 16)]), BCAP)
        off = b * BSTR + cb
        so_v[pl.ds(off, 16)] = _splat(s)
        do_v[pl.ds(off, 16)] = _splat(d - b * NLOC)
        cn_v[pl.ds(b * 16, 16)] = _splat(cb + 1)
        return 0
    lax.fori_loop(0, EPT, edge, 0)

    pltpu.sync_copy(so_v, souts_hbm.at[w])
    pltpu.sync_copy(do_v, douts_hbm.at[w])
    pltpu.sync_copy(cn_v, cnts_hbm.at[w])


_bin_edges = pl.kernel(
    _bin_body,
    out_type=[_SDS((NW, NW * BSTR), jnp.int32),
              _SDS((NW, NW * BSTR), jnp.int32),
              _SDS((NW, NW * 16), jnp.int32)],
    mesh=_mesh,
    compiler_params=pltpu.CompilerParams(use_tc_tiling_on_sc=False),
    scratch_types=[pltpu.VMEM((EPT + 16,), jnp.int32),
                   pltpu.VMEM((EPT + 16,), jnp.int32),
                   pltpu.VMEM((NW * BSTR,), jnp.int32),
                   pltpu.VMEM((NW * BSTR,), jnp.int32),
                   pltpu.VMEM((NW * 16,), jnp.int32)],
)


SSTR = 96
SCAP = 80
GD = 4
GW = GD * SSTR
NG = NLOC // GD


def _sort_body(souts_hbm, douts_hbm, cnts_hbm, slist_hbm, scnt_hbm,
               inbox_s, inbox_d, cbuf, slv, scv):
    w = _wid()

    pltpu.sync_copy(cnts_hbm.at[:, pl.ds(w * 16, 16)], cbuf)
    pltpu.sync_copy(souts_hbm.at[:, pl.ds(w * BSTR, BSTR)], inbox_s)
    pltpu.sync_copy(douts_hbm.at[:, pl.ds(w * BSTR, BSTR)], inbox_d)

    def zs(i, _):
        slv[pl.ds(i * 16, 16)] = _splat(N)
        return 0
    lax.fori_loop(0, NLOC * SSTR // 16, zs, 0)

    def zc(i, _):
        scv[pl.ds(i * 16, 16)] = jnp.zeros((16,), jnp.int32)
        return 0
    lax.fori_loop(0, NLOC, zc, 0)

    def seg(t, _):
        cnt = jnp.minimum(_lane0(cbuf[t, pl.ds(0, 16)]), BCAP)

        def edge(e, _):
            dl = _lane0(inbox_d[t, pl.ds(e, 16)])
            s = _lane0(inbox_s[t, pl.ds(e, 16)])
            c = jnp.minimum(_lane0(scv[pl.ds(dl * 16, 16)]), SCAP - 1)
            slv[pl.ds(dl * SSTR + c, 16)] = _splat(s)
            scv[pl.ds(dl * 16, 16)] = _splat(c + 1)
            return 0
        lax.fori_loop(0, cnt, edge, 0)
        return 0
    lax.fori_loop(0, NW, seg, 0)

    def clean(dl, _):
        c = jnp.minimum(_lane0(scv[pl.ds(dl * 16, 16)]), SCAP)
        slv[pl.ds(dl * SSTR + c, 16)] = _splat(N)
        return 0
    lax.fori_loop(0, NLOC, clean, 0)

    pltpu.sync_copy(slv, slist_hbm.at[w])
    pltpu.sync_copy(scv, scnt_hbm.at[w])


_sort_edges = pl.kernel(
    _sort_body,
    out_type=[_SDS((NW, NLOC * SSTR), jnp.int32),
              _SDS((NW, NLOC * 16), jnp.int32)],
    mesh=_mesh,
    compiler_params=pltpu.CompilerParams(use_tc_tiling_on_sc=False),
    scratch_types=[pltpu.VMEM((NW, BSTR), jnp.int32),
                   pltpu.VMEM((NW, BSTR), jnp.int32),
                   pltpu.VMEM((NW, 16), jnp.int32),
                   pltpu.VMEM((NLOC * SSTR,), jnp.int32),
                   pltpu.VMEM((NLOC * 16,), jnp.int32)],
)


def _segmax_body(D, b_hbm, slist_hbm, m_hbm,
                 slv, rows0, rows1, outb, sem0, sem1):
    w = _wid()
    pltpu.sync_copy(slist_hbm.at[w], slv)

    def issue(g, rows, sem):
        base = g * GW
        for c in range(GW // 128):
            pltpu.async_copy(b_hbm.at[slv.at[pl.ds(base + c * 128, 128)]],
                             rows.at[pl.ds(c * 128, 128)], sem)

    def drain(g, rows, sem):
        base = g * GW
        for c in range(GW // 128):
            pltpu.make_async_copy(b_hbm.at[slv.at[pl.ds(base + c * 128, 128)]],
                                  rows.at[pl.ds(c * 128, 128)], sem).wait()

    ninf = jnp.full((16,), -jnp.inf, jnp.float32)

    def _tree16(rows, base, j):
        sl = pl.ds(j * 16, 16)
        vs = [rows[base + u, sl] for u in range(16)]
        while len(vs) > 1:
            vs = [jnp.maximum(vs[2 * i], vs[2 * i + 1])
                  for i in range(len(vs) // 2)]
        return vs[0]

    def process(g, rows):
        for k in range(GD):
            dl = g * GD + k
            for j in range(D // 16):
                outb[dl, pl.ds(j * 16, 16)] = ninf

            def blk(b, _):
                base = k * SSTR + b * 16
                for j in range(D // 16):
                    sl = pl.ds(j * 16, 16)
                    outb[dl, sl] = jnp.maximum(outb[dl, sl],
                                               _tree16(rows, base, j))
                return 0
            lax.fori_loop(0, SSTR // 16, blk, 0)

    issue(0, rows0, sem0)
    issue(1, rows1, sem1)

    def pair(i, _):
        g0 = 2 * i
        drain(g0, rows0, sem0)
        process(g0, rows0)

        @pl.when(g0 + 2 < NG)
        def _():
            issue(g0 + 2, rows0, sem0)

        g1 = g0 + 1
        drain(g1, rows1, sem1)
        process(g1, rows1)

        @pl.when(g1 + 2 < NG)
        def _():
            issue(g1 + 2, rows1, sem1)
        return 0
    lax.fori_loop(0, NG // 2, pair, 0)

    @pl.when(w < NW - 1)
    def _():
        pltpu.sync_copy(outb, m_hbm.at[pl.ds(w * NLOC, NLOC)])

    @pl.when(w == NW - 1)
    def _():
        pltpu.sync_copy(outb.at[pl.ds(0, N - (NW - 1) * NLOC)],
                        m_hbm.at[pl.ds((NW - 1) * NLOC, N - (NW - 1) * NLOC)])


def _make_segmax(D):
    return pl.kernel(
        functools.partial(_segmax_body, D),
        out_type=_SDS((N, D), jnp.float32),
        mesh=_mesh,
        compiler_params=pltpu.CompilerParams(use_tc_tiling_on_sc=False),
        scratch_types=[pltpu.VMEM((NLOC * SSTR,), jnp.int32),
                       pltpu.VMEM((GW, D), jnp.float32),
                       pltpu.VMEM((GW, D), jnp.float32),
                       pltpu.VMEM((NLOC, D), jnp.float32),
                       pltpu.SemaphoreType.DMA,
                       pltpu.SemaphoreType.DMA],
    )


_segmax32 = _make_segmax(32)
_segmax64 = _make_segmax(64)


def _elu(v):
    return jnp.where(v > 0, v, jnp.exp(v) - 1.0)


def _stage1_body(x_ref, wa_ref, wb_ref, a_ref, b_ref):
    x = x_ref[...]
    a_ref[...] = jnp.dot(x, wa_ref[...], preferred_element_type=jnp.float32)
    b_ref[...] = jnp.dot(x, wb_ref[...], preferred_element_type=jnp.float32)


def _stage_body(a_ref, m_ref, bias_ref, wa_ref, wb_ref, a2_ref, b2_ref):
    m = m_ref[...]
    h = a_ref[...] + m + bias_ref[...]
    h = jnp.where(m > -3e38, _elu(h), 0.0)
    a2_ref[...] = jnp.dot(h, wa_ref[...], preferred_element_type=jnp.float32)
    b2_ref[...] = jnp.dot(h, wb_ref[...], preferred_element_type=jnp.float32)


def _final_body(a_ref, m_ref, bias_ref, w1_ref, b1_ref, w2_ref, b2_ref,
                w3_ref, b3_ref, o_ref, gm):
    i = pl.program_id(0)
    m = m_ref[...]
    h = a_ref[...] + m + bias_ref[...]
    h = jnp.where(m > -3e38, _elu(h), 0.0)
    bm = jnp.max(h, axis=0, keepdims=True)

    @pl.when(i == 0)
    def _():
        gm[0:1, :] = bm

    @pl.when(i > 0)
    def _():
        gm[0:1, :] = jnp.maximum(gm[0:1, :], bm)

    @pl.when(i == pl.num_programs(0) - 1)
    def _():
        g = gm[0:1, :]
        g = _elu(jnp.dot(g, w1_ref[...], preferred_element_type=jnp.float32)
                 + b1_ref[...])
        g = _elu(jnp.dot(g, w2_ref[...], preferred_element_type=jnp.float32)
                 + b2_ref[...])
        g = (jnp.dot(g, w3_ref[...], preferred_element_type=jnp.float32)
             + b3_ref[...])
        z = g - jnp.max(g, axis=1, keepdims=True)
        o_ref[...] = z - jnp.log(jnp.sum(jnp.exp(z), axis=1, keepdims=True))


_BN = 2000


def _full(shape):
    return pl.BlockSpec(shape, lambda i: (0, 0))


def _rows(d):
    return pl.BlockSpec((_BN, d), lambda i: (i, 0))


def _stage1(x, wa, wb):
    return pl.pallas_call(
        _stage1_body,
        grid=(N // _BN,),
        in_specs=[_rows(F_IN), _full((F_IN, 32)), _full((F_IN, 32))],
        out_specs=[_rows(32), _rows(32)],
        out_shape=[_SDS((N, 32), jnp.float32), _SDS((N, 32), jnp.float32)],
    )(x, wa, wb)


def _stage(a, m, bias, wa, wb, din, dout):
    return pl.pallas_call(
        _stage_body,
        grid=(N // _BN,),
        in_specs=[_rows(din), _rows(din), _full((1, din)),
                  _full((din, dout)), _full((din, dout))],
        out_specs=[_rows(dout), _rows(dout)],
        out_shape=[_SDS((N, dout), jnp.float32),
                   _SDS((N, dout), jnp.float32)],
    )(a, m, bias, wa, wb)


def _final(a, m, bias, w1, b1, w2, b2, w3, b3):
    return pl.pallas_call(
        _final_body,
        grid=(N // _BN,),
        in_specs=[_rows(64), _rows(64), _full((1, 64)),
                  _full((64, 64)), _full((1, 64)),
                  _full((64, 32)), _full((1, 32)),
                  _full((32, 10)), _full((1, 10))],
        out_specs=pl.BlockSpec((1, 10), lambda i: (0, 0)),
        out_shape=_SDS((1, 10), jnp.float32),
        scratch_shapes=[pltpu.VMEM((8, 64), jnp.float32)],
    )(a, m, bias, w1, b1, w2, b2, w3, b3)


def kernel(x, edge_index, Wt1, bt1, Wp1, bp1, bn_g, bn_b, bn_m, bn_v,
           Wt2, bt2, Wp2, bp2, Wt3, bt3, Wp3, bp3,
           fc1_W, fc1_b, fc2_W, fc2_b, fc3_W, fc3_b):
    s1 = bn_g / jnp.sqrt(bn_v + 1e-5)
    sh1 = bn_b - bn_m * s1
    wa1 = Wt1 * s1
    wb1 = (Wp1 - Wt1) * s1
    b1 = ((bt1 + bp1) * s1 + sh1).reshape(1, 32)
    wa2, wb2, b2 = Wt2, Wp2 - Wt2, (bt2 + bp2).reshape(1, 64)
    wa3, wb3, b3 = Wt3, Wp3 - Wt3, (bt3 + bp3).reshape(1, 64)

    souts, douts, cnts = _bin_edges(edge_index.reshape(2 * E))
    slist, _ = _sort_edges(souts, douts, cnts)

    def pad(bb, d):
        return jnp.concatenate([bb, jnp.full((8, d), -jnp.inf, jnp.float32)])

    a1, bb1 = _stage1(x, wa1, wb1)
    m1 = _segmax32(pad(bb1, 32), slist)
    a2, bb2 = _stage(a1, m1, b1, wa2, wb2, 32, 64)
    m2 = _segmax64(pad(bb2, 64), slist)
    a3, bb3 = _stage(a2, m2, b2, wa3, wb3, 64, 64)
    m3 = _segmax64(pad(bb3, 64), slist)
    return _final(a3, m3, b3, fc1_W, fc1_b.reshape(1, 64),
                  fc2_W, fc2_b.reshape(1, 32), fc3_W, fc3_b.reshape(1, 10))


# speedup vs baseline: 14.2531x; 14.2531x over previous
import functools

import jax
import jax.numpy as jnp
from jax import lax
from jax.experimental import pallas as pl
from jax.experimental.pallas import tpu as pltpu
from jax.experimental.pallas import tpu_sc as plsc

N = 10000
E = 320000
F_IN = 128

NC = 2
NS = 16
NW = NC * NS
NLOC = 320
EPT = E // NW
BSTR = 528
BCAP = 512

_SDS = jax.ShapeDtypeStruct
_mesh = plsc.VectorSubcoreMesh(core_axis_name="c", subcore_axis_name="s")


def _wid():
    return lax.axis_index("s") * NC + lax.axis_index("c")


def _lane0(v):
    return lax.squeeze(lax.slice(v, (0,), (1,)), (0,))


def _splat(s):
    return jnp.full((16,), s, jnp.int32)


def _bin_body(ef_hbm, souts_hbm, douts_hbm, cnts_hbm,
              sbuf, dbuf, so_v, do_v, cn_v):
    w = _wid()
    e0 = w * EPT

    pltpu.sync_copy(ef_hbm.at[pl.ds(e0, EPT)], sbuf.at[pl.ds(0, EPT)])
    pltpu.sync_copy(ef_hbm.at[pl.ds(E + e0, EPT)], dbuf.at[pl.ds(0, EPT)])

    def z16(i, _):
        cn_v[pl.ds(i * 16, 16)] = jnp.zeros((16,), jnp.int32)
        return 0
    lax.fori_loop(0, NW, z16, 0)

    def zs(i, _):
        so_v[pl.ds(i * 16, 16)] = jnp.zeros((16,), jnp.int32)
        return 0
    lax.fori_loop(0, NW * BSTR // 16, zs, 0)

    def edge(e, _):
        d = _lane0(dbuf[pl.ds(e, 16)])
        s = _lane0(sbuf[pl.ds(e, 16)])
        b = (d * 6554) >> 21
        cb = jnp.minimum(_lane0(cn_v[pl.ds(b * 16, 16)]), BCAP)
        off = b * BSTR + cb
        so_v[pl.ds(off, 16)] = _splat(s)
        do_v[pl.ds(off, 16)] = _splat(d - b * NLOC)
        cn_v[pl.ds(b * 16, 16)] = _splat(cb + 1)
        return 0
    lax.fori_loop(0, EPT, edge, 0)

    pltpu.sync_copy(so_v, souts_hbm.at[w])
    pltpu.sync_copy(do_v, douts_hbm.at[w])
    pltpu.sync_copy(cn_v, cnts_hbm.at[w])


_bin_edges = pl.kernel(
    _bin_body,
    out_type=[_SDS((NW, NW * BSTR), jnp.int32),
              _SDS((NW, NW * BSTR), jnp.int32),
              _SDS((NW, NW * 16), jnp.int32)],
    mesh=_mesh,
    compiler_params=pltpu.CompilerParams(use_tc_tiling_on_sc=False),
    scratch_types=[pltpu.VMEM((EPT + 16,), jnp.int32),
                   pltpu.VMEM((EPT + 16,), jnp.int32),
                   pltpu.VMEM((NW * BSTR,), jnp.int32),
                   pltpu.VMEM((NW * BSTR,), jnp.int32),
                   pltpu.VMEM((NW * 16,), jnp.int32)],
)


SSTR = 96
SCAP = 80
GD = 4
GW = GD * SSTR
NG = NLOC // GD


def _sort_body(souts_hbm, douts_hbm, cnts_hbm, slist_hbm, scnt_hbm,
               inbox_s, inbox_d, cbuf, slv, scv):
    w = _wid()

    pltpu.sync_copy(cnts_hbm.at[:, pl.ds(w * 16, 16)], cbuf)
    pltpu.sync_copy(souts_hbm.at[:, pl.ds(w * BSTR, BSTR)], inbox_s)
    pltpu.sync_copy(douts_hbm.at[:, pl.ds(w * BSTR, BSTR)], inbox_d)

    def _sent(pos):
        return _splat(N) + ((_splat(pos) + lax.iota(jnp.int32, 16)) & 63)

    def zs(i, _):
        slv[pl.ds(i * 16, 16)] = _sent(i * 16)
        return 0
    lax.fori_loop(0, NLOC * SSTR // 16, zs, 0)

    def zc(i, _):
        scv[pl.ds(i * 16, 16)] = jnp.zeros((16,), jnp.int32)
        return 0
    lax.fori_loop(0, NLOC, zc, 0)

    def seg(t, _):
        cnt = jnp.minimum(_lane0(cbuf[t, pl.ds(0, 16)]), BCAP)

        def edge(e, _):
            dl = _lane0(inbox_d[t, pl.ds(e, 16)])
            s = _lane0(inbox_s[t, pl.ds(e, 16)])
            c = jnp.minimum(_lane0(scv[pl.ds(dl * 16, 16)]), SCAP - 1)
            slv[pl.ds(dl * SSTR + c, 16)] = _splat(s)
            scv[pl.ds(dl * 16, 16)] = _splat(c + 1)
            return 0
        lax.fori_loop(0, cnt, edge, 0)
        return 0
    lax.fori_loop(0, NW, seg, 0)

    def clean(dl, _):
        c = jnp.minimum(_lane0(scv[pl.ds(dl * 16, 16)]), SCAP)
        slv[pl.ds(dl * SSTR + c, 16)] = _sent(dl * SSTR + c)
        return 0
    lax.fori_loop(0, NLOC, clean, 0)

    pltpu.sync_copy(slv, slist_hbm.at[w])
    pltpu.sync_copy(scv, scnt_hbm.at[w])


_sort_edges = pl.kernel(
    _sort_body,
    out_type=[_SDS((NW, NLOC * SSTR), jnp.int32),
              _SDS((NW, NLOC * 16), jnp.int32)],
    mesh=_mesh,
    compiler_params=pltpu.CompilerParams(use_tc_tiling_on_sc=False),
    scratch_types=[pltpu.VMEM((NW, BSTR), jnp.int32),
                   pltpu.VMEM((NW, BSTR), jnp.int32),
                   pltpu.VMEM((NW, 16), jnp.int32),
                   pltpu.VMEM((NLOC * SSTR,), jnp.int32),
                   pltpu.VMEM((NLOC * 16,), jnp.int32)],
)


def _segmax_body(D, b_hbm, slist_hbm, m_hbm,
                 slv, rows0, rows1, outb, sem0, sem1):
    w = _wid()
    pltpu.sync_copy(slist_hbm.at[w], slv)

    def issue(g, rows, sem):
        base = g * GW
        for c in range(GW // 128):
            pltpu.async_copy(b_hbm.at[slv.at[pl.ds(base + c * 128, 128)]],
                             rows.at[pl.ds(c * 128, 128)], sem)

    def drain(g, rows, sem):
        base = g * GW
        for c in range(GW // 128):
            pltpu.make_async_copy(b_hbm.at[slv.at[pl.ds(base + c * 128, 128)]],
                                  rows.at[pl.ds(c * 128, 128)], sem).wait()

    ninf = jnp.full((16,), -jnp.inf, jnp.float32)

    def _tree16(rows, base, j):
        sl = pl.ds(j * 16, 16)
        vs = [rows[base + u, sl] for u in range(16)]
        while len(vs) > 1:
            vs = [jnp.maximum(vs[2 * i], vs[2 * i + 1])
                  for i in range(len(vs) // 2)]
        return vs[0]

    def process(g, rows):
        for k in range(GD):
            dl = g * GD + k
            for j in range(D // 16):
                outb[dl, pl.ds(j * 16, 16)] = ninf

            def blk(b, _):
                base = k * SSTR + b * 16
                for j in range(D // 16):
                    sl = pl.ds(j * 16, 16)
                    outb[dl, sl] = jnp.maximum(outb[dl, sl],
                                               _tree16(rows, base, j))
                return 0
            lax.fori_loop(0, SSTR // 16, blk, 0)

    issue(0, rows0, sem0)
    issue(1, rows1, sem1)

    def pair(i, _):
        g0 = 2 * i
        drain(g0, rows0, sem0)
        process(g0, rows0)

        @pl.when(g0 + 2 < NG)
        def _():
            issue(g0 + 2, rows0, sem0)

        g1 = g0 + 1
        drain(g1, rows1, sem1)
        process(g1, rows1)

        @pl.when(g1 + 2 < NG)
        def _():
            issue(g1 + 2, rows1, sem1)
        return 0
    lax.fori_loop(0, NG // 2, pair, 0)

    @pl.when(w < NW - 1)
    def _():
        pltpu.sync_copy(outb, m_hbm.at[pl.ds(w * NLOC, NLOC)])

    @pl.when(w == NW - 1)
    def _():
        pltpu.sync_copy(outb.at[pl.ds(0, N - (NW - 1) * NLOC)],
                        m_hbm.at[pl.ds((NW - 1) * NLOC, N - (NW - 1) * NLOC)])


def _make_segmax(D):
    return pl.kernel(
        functools.partial(_segmax_body, D),
        out_type=_SDS((N, D), jnp.float32),
        mesh=_mesh,
        compiler_params=pltpu.CompilerParams(use_tc_tiling_on_sc=False),
        scratch_types=[pltpu.VMEM((NLOC * SSTR,), jnp.int32),
                       pltpu.VMEM((GW, D), jnp.float32),
                       pltpu.VMEM((GW, D), jnp.float32),
                       pltpu.VMEM((NLOC, D), jnp.float32),
                       pltpu.SemaphoreType.DMA,
                       pltpu.SemaphoreType.DMA],
    )


_segmax32 = _make_segmax(32)
_segmax64 = _make_segmax(64)


def _elu(v):
    return jnp.where(v > 0, v, jnp.exp(v) - 1.0)


def _stage1_body(x_ref, wa_ref, wb_ref, a_ref, b_ref):
    x = x_ref[...]
    a_ref[...] = jnp.dot(x, wa_ref[...], preferred_element_type=jnp.float32)
    b_ref[...] = jnp.dot(x, wb_ref[...], preferred_element_type=jnp.float32)


def _stage_body(a_ref, m_ref, bias_ref, wa_ref, wb_ref, a2_ref, b2_ref):
    m = m_ref[...]
    h = a_ref[...] + m + bias_ref[...]
    h = jnp.where(m > -3e38, _elu(h), 0.0)
    a2_ref[...] = jnp.dot(h, wa_ref[...], preferred_element_type=jnp.float32)
    b2_ref[...] = jnp.dot(h, wb_ref[...], preferred_element_type=jnp.float32)


def _final_body(a_ref, m_ref, bias_ref, w1_ref, b1_ref, w2_ref, b2_ref,
                w3_ref, b3_ref, o_ref, gm):
    i = pl.program_id(0)
    m = m_ref[...]
    h = a_ref[...] + m + bias_ref[...]
    h = jnp.where(m > -3e38, _elu(h), 0.0)
    bm = jnp.max(h, axis=0, keepdims=True)

    @pl.when(i == 0)
    def _():
        gm[0:1, :] = bm

    @pl.when(i > 0)
    def _():
        gm[0:1, :] = jnp.maximum(gm[0:1, :], bm)

    @pl.when(i == pl.num_programs(0) - 1)
    def _():
        g = gm[0:1, :]
        g = _elu(jnp.dot(g, w1_ref[...], preferred_element_type=jnp.float32)
                 + b1_ref[...])
        g = _elu(jnp.dot(g, w2_ref[...], preferred_element_type=jnp.float32)
                 + b2_ref[...])
        g = (jnp.dot(g, w3_ref[...], preferred_element_type=jnp.float32)
             + b3_ref[...])
        z = g - jnp.max(g, axis=1, keepdims=True)
        o_ref[...] = z - jnp.log(jnp.sum(jnp.exp(z), axis=1, keepdims=True))


_BN = 2000


def _full(shape):
    return pl.BlockSpec(shape, lambda i: (0, 0))


def _rows(d):
    return pl.BlockSpec((_BN, d), lambda i: (i, 0))


def _stage1(x, wa, wb):
    return pl.pallas_call(
        _stage1_body,
        grid=(N // _BN,),
        in_specs=[_rows(F_IN), _full((F_IN, 32)), _full((F_IN, 32))],
        out_specs=[_rows(32), _rows(32)],
        out_shape=[_SDS((N, 32), jnp.float32), _SDS((N, 32), jnp.float32)],
    )(x, wa, wb)


def _stage(a, m, bias, wa, wb, din, dout):
    return pl.pallas_call(
        _stage_body,
        grid=(N // _BN,),
        in_specs=[_rows(din), _rows(din), _full((1, din)),
                  _full((din, dout)), _full((din, dout))],
        out_specs=[_rows(dout), _rows(dout)],
        out_shape=[_SDS((N, dout), jnp.float32),
                   _SDS((N, dout), jnp.float32)],
    )(a, m, bias, wa, wb)


def _final(a, m, bias, w1, b1, w2, b2, w3, b3):
    return pl.pallas_call(
        _final_body,
        grid=(N // _BN,),
        in_specs=[_rows(64), _rows(64), _full((1, 64)),
                  _full((64, 64)), _full((1, 64)),
                  _full((64, 32)), _full((1, 32)),
                  _full((32, 10)), _full((1, 10))],
        out_specs=pl.BlockSpec((1, 10), lambda i: (0, 0)),
        out_shape=_SDS((1, 10), jnp.float32),
        scratch_shapes=[pltpu.VMEM((8, 64), jnp.float32)],
    )(a, m, bias, w1, b1, w2, b2, w3, b3)


def kernel(x, edge_index, Wt1, bt1, Wp1, bp1, bn_g, bn_b, bn_m, bn_v,
           Wt2, bt2, Wp2, bp2, Wt3, bt3, Wp3, bp3,
           fc1_W, fc1_b, fc2_W, fc2_b, fc3_W, fc3_b):
    s1 = bn_g / jnp.sqrt(bn_v + 1e-5)
    sh1 = bn_b - bn_m * s1
    wa1 = Wt1 * s1
    wb1 = (Wp1 - Wt1) * s1
    b1 = ((bt1 + bp1) * s1 + sh1).reshape(1, 32)
    wa2, wb2, b2 = Wt2, Wp2 - Wt2, (bt2 + bp2).reshape(1, 64)
    wa3, wb3, b3 = Wt3, Wp3 - Wt3, (bt3 + bp3).reshape(1, 64)

    souts, douts, cnts = _bin_edges(edge_index.reshape(2 * E))
    slist, _ = _sort_edges(souts, douts, cnts)

    def pad(bb, d):
        return jnp.concatenate([bb, jnp.full((64, d), -jnp.inf, jnp.float32)])

    a1, bb1 = _stage1(x, wa1, wb1)
    m1 = _segmax32(pad(bb1, 32), slist)
    a2, bb2 = _stage(a1, m1, b1, wa2, wb2, 32, 64)
    m2 = _segmax64(pad(bb2, 64), slist)
    a3, bb3 = _stage(a2, m2, b2, wa3, wb3, 64, 64)
    m3 = _segmax64(pad(bb3, 64), slist)
    return _final(a3, m3, b3, fc1_W, fc1_b.reshape(1, 64),
                  fc2_W, fc2_b.reshape(1, 32), fc3_W, fc3_b.reshape(1, 10))
